# baseline (speedup 1.0000x reference)
import jax
import jax.numpy as jnp
from jax.experimental import pallas as pl
from jax.experimental.pallas import tpu as pltpu

N_POI, N_EDGE, D = 16384, 4096, 128
BM_A = 256
BM_B = 512
A_STEPS = N_EDGE // BM_A
B_STEPS = N_POI // BM_B

_PREC = jax.lax.Precision.DEFAULT


def _merged_kernel(hg_a_ref, poi_ref, edge_ref, wp_ref, we_ref, wf_ref,
                   hg_b_ref, prop_ref, fe_ref):
    i = pl.program_id(0)

    @pl.when(i < A_STEPS)
    def _phase_a():
        t = jnp.dot(hg_a_ref[...], poi_ref[...],
                    preferred_element_type=jnp.float32, precision=_PREC)
        w1 = jnp.dot(wp_ref[...], wf_ref[:D, :],
                     preferred_element_type=jnp.float32, precision=_PREC)
        w2 = jnp.dot(we_ref[...], wf_ref[D:, :],
                     preferred_element_type=jnp.float32, precision=_PREC)
        fe = (jnp.dot(t, w1, preferred_element_type=jnp.float32,
                      precision=_PREC)
              + jnp.dot(edge_ref[...], w2, preferred_element_type=jnp.float32,
                        precision=_PREC))
        fe_ref[pl.ds(i * BM_A, BM_A), :] = fe

    @pl.when(i >= A_STEPS)
    def _phase_b():
        prop_ref[...] = jnp.dot(hg_b_ref[...], fe_ref[...],
                                preferred_element_type=jnp.float32,
                                precision=_PREC)


def kernel(poi_embs, edge_embs, hg_edge_to_poi, hg_poi_to_edge,
           W_poi, W_edge, W_fusion):
    propagated_poi, fused_edge = pl.pallas_call(
        _merged_kernel,
        grid=(A_STEPS + B_STEPS,),
        in_specs=[
            pl.BlockSpec((BM_A, N_POI),
                         lambda i: (jnp.minimum(i, A_STEPS - 1), 0)),
            pl.BlockSpec((N_POI, D), lambda i: (0, 0)),
            pl.BlockSpec((BM_A, D),
                         lambda i: (jnp.minimum(i, A_STEPS - 1), 0)),
            pl.BlockSpec((D, D), lambda i: (0, 0)),
            pl.BlockSpec((D, D), lambda i: (0, 0)),
            pl.BlockSpec((2 * D, D), lambda i: (0, 0)),
            pl.BlockSpec((BM_B, N_EDGE),
                         lambda i: (jnp.maximum(i - A_STEPS, 0), 0)),
        ],
        out_specs=[
            pl.BlockSpec((BM_B, D),
                         lambda i: (jnp.maximum(i - A_STEPS, 0), 0)),
            pl.BlockSpec((N_EDGE, D), lambda i: (0, 0)),
        ],
        out_shape=[
            jax.ShapeDtypeStruct((N_POI, D), jnp.float32),
            jax.ShapeDtypeStruct((N_EDGE, D), jnp.float32),
        ],
        compiler_params=pltpu.CompilerParams(
            dimension_semantics=("arbitrary",),
            vmem_limit_bytes=67108864),
    )(hg_poi_to_edge, poi_embs, edge_embs, W_poi, W_edge, W_fusion,
      hg_edge_to_poi)

    return propagated_poi, fused_edge


# speedup vs baseline: 1.0202x; 1.0202x over previous
import jax
import jax.numpy as jnp
from jax.experimental import pallas as pl
from jax.experimental.pallas import tpu as pltpu

N_POI, N_EDGE, D = 16384, 4096, 128
BM_A = 256
BM_B = 512
A_STEPS = N_EDGE // BM_A
B_STEPS = N_POI // BM_B
KA = N_POI // 2
KB = N_EDGE // 2

_PREC = jax.lax.Precision.DEFAULT


def _dot(a, b):
    return jnp.dot(a, b, preferred_element_type=jnp.float32, precision=_PREC)


def _merged_kernel(hg_a1_ref, hg_a2_ref, poi_ref, edge_ref,
                   wp_ref, we_ref, wf_ref, hg_b1_ref, hg_b2_ref,
                   prop_ref, fe_ref):
    i = pl.program_id(0)

    @pl.when(i < A_STEPS)
    def _phase_a():
        t = _dot(hg_a1_ref[...], poi_ref[:KA, :]) + _dot(
            hg_a2_ref[...], poi_ref[KA:, :])
        w1 = _dot(wp_ref[...], wf_ref[:D, :])
        w2 = _dot(we_ref[...], wf_ref[D:, :])
        fe_ref[pl.ds(i * BM_A, BM_A), :] = (
            _dot(t, w1) + _dot(edge_ref[...], w2))

    @pl.when(i >= A_STEPS)
    def _phase_b():
        prop_ref[...] = _dot(hg_b1_ref[...], fe_ref[:KB, :]) + _dot(
            hg_b2_ref[...], fe_ref[KB:, :])


def kernel(poi_embs, edge_embs, hg_edge_to_poi, hg_poi_to_edge,
           W_poi, W_edge, W_fusion):
    a_idx = lambda i: (jnp.minimum(i, A_STEPS - 1), 0)
    a_idx2 = lambda i: (jnp.minimum(i, A_STEPS - 1), 1)
    b_idx = lambda i: (jnp.maximum(i - A_STEPS, 0), 0)
    b_idx2 = lambda i: (jnp.maximum(i - A_STEPS, 0), 1)
    propagated_poi, fused_edge = pl.pallas_call(
        _merged_kernel,
        grid=(A_STEPS + B_STEPS,),
        in_specs=[
            pl.BlockSpec((BM_A, KA), a_idx),
            pl.BlockSpec((BM_A, KA), a_idx2),
            pl.BlockSpec((N_POI, D), lambda i: (0, 0)),
            pl.BlockSpec((BM_A, D), a_idx),
            pl.BlockSpec((D, D), lambda i: (0, 0)),
            pl.BlockSpec((D, D), lambda i: (0, 0)),
            pl.BlockSpec((2 * D, D), lambda i: (0, 0)),
            pl.BlockSpec((BM_B, KB), b_idx),
            pl.BlockSpec((BM_B, KB), b_idx2),
        ],
        out_specs=[
            pl.BlockSpec((BM_B, D), b_idx),
            pl.BlockSpec((N_EDGE, D), lambda i: (0, 0)),
        ],
        out_shape=[
            jax.ShapeDtypeStruct((N_POI, D), jnp.float32),
            jax.ShapeDtypeStruct((N_EDGE, D), jnp.float32),
        ],
        compiler_params=pltpu.CompilerParams(
            dimension_semantics=("arbitrary",),
            vmem_limit_bytes=67108864),
    )(hg_poi_to_edge, hg_poi_to_edge, poi_embs, edge_embs,
      W_poi, W_edge, W_fusion, hg_edge_to_poi, hg_edge_to_poi)

    return propagated_poi, fused_edge
